# Initial kernel scaffold; baseline (speedup 1.0000x reference)
import jax
import jax.numpy as jnp
from jax.experimental import pallas as pl


def kernel(x, edge_index, edge_weight, p, W_ih, W_hh, b_ih, b_hh, W_gcn, W_lin, b_lin):
    raise NotImplementedError("write your pallas kernel here")


# speedup vs baseline: 22.0394x; 22.0394x over previous
import functools

import jax
import jax.numpy as jnp
from jax import lax
from jax.experimental import pallas as pl
from jax.experimental.pallas import tpu as pltpu
from jax.experimental.pallas import tpu_sc as plsc

N = 10000
D = 128
K = 128
NB = 80
NPAD = NB * 128
EROWS = 2560
EPAD = EROWS * 128
RPW = EROWS // 32
NACC = 10112
NRW = NACC // 16
CCAP = 256
CTOT = 384
BIG_NEG = -3.0e38


HI = lax.Precision.HIGHEST


def _score_body(x3_ref, p_ref, nrm_ref, sval_ref, slot_ref):
    x3 = x3_ref[...].astype(jnp.bfloat16)
    pp = p_ref[...].astype(jnp.bfloat16)
    s3 = lax.dot_general(x3, pp, (((2,), (1,)), ((), ())),
                         preferred_element_type=jnp.float32)
    s = s3[:, :, 0] / nrm_ref[...]
    ii = lax.broadcasted_iota(jnp.int32, (NB, D), 0)
    jj = lax.broadcasted_iota(jnp.int32, (NB, D), 1)
    valid = ii * D + jj < N
    s = jnp.where(valid, s, -jnp.inf)
    imin = jnp.int32(-(2 ** 31))
    b = lax.bitcast_convert_type(s, jnp.int32)
    skey = b ^ (lax.shift_right_arithmetic(b, 31) & jnp.int32(0x7FFFFFFF))
    skey = jnp.where(valid, skey, imin)

    def step(i, pref):
        cand = pref | lax.shift_left(jnp.int32(1), 31 - i)
        scand = cand ^ imin
        cnt = jnp.sum((skey >= scand).astype(jnp.int32))
        return jnp.where(cnt >= K, cand, pref)

    pref = lax.fori_loop(0, 32, step, jnp.int32(0))
    thr_f = lax.bitcast_convert_type(pref ^ imin, jnp.float32)
    flag = s >= thr_f
    flag_f = jnp.where(flag, 1.0, 0.0).astype(jnp.float32)
    u0 = lax.broadcasted_iota(jnp.int32, (D, D), 0)
    u1 = lax.broadcasted_iota(jnp.int32, (D, D), 1)
    tri_u = jnp.where(u0 < u1, 1.0, 0.0).astype(jnp.float32)
    v0 = lax.broadcasted_iota(jnp.int32, (NB, NB), 0)
    v1 = lax.broadcasted_iota(jnp.int32, (NB, NB), 1)
    tri_l = jnp.where(v1 < v0, 1.0, 0.0).astype(jnp.float32)
    dnum = (((1,), (0,)), ((), ()))
    ex_lane = lax.dot_general(flag_f, tri_u, dnum, precision=HI,
                              preferred_element_type=jnp.float32)
    ones_col = jnp.full((D, 1), 1.0, jnp.float32)
    row_tot = lax.dot_general(flag_f, ones_col, dnum, precision=HI,
                              preferred_element_type=jnp.float32)
    row_pre = lax.dot_general(tri_l, row_tot, dnum, precision=HI,
                              preferred_element_type=jnp.float32)
    pos = ex_lane + row_pre
    slot = jnp.where(flag, jnp.minimum(pos, float(CCAP - 1)).astype(jnp.int32),
                     CCAP + jj)
    sval_ref[...] = s
    slot_ref[...] = slot


def _scores_and_slots(x3, p2, nrm_b):
    return pl.pallas_call(
        _score_body,
        out_shape=[
            jax.ShapeDtypeStruct((NB, D), jnp.float32),
            jax.ShapeDtypeStruct((NB, D), jnp.int32),
        ],
    )(x3, p2, nrm_b)


def _select_body(sval_hbm, slot_hbm, x_hbm, dst2_hbm, w2_hbm,
                 xsel_out, tvals_out, degp_out,
                 dst_t, w_t, zbuf, svalv, slotv, gidb, onesb,
                 cval_t, cgid_t, cnt_t, ridx, tbuf, vbuf, gbuf, dbuf,
                 permf, permi, xrows, sh_deg, sh_cval, sh_cgid, sh_cnt,
                 sh_tvals, sh_perm, gsem):
    core = lax.axis_index("c")
    sub = lax.axis_index("s")
    wid = sub * 2 + core
    lane = lax.iota(jnp.int32, 16)
    lane_f = lane.astype(jnp.float32)
    one = jnp.full((16,), 1.0, jnp.float32)
    zero = jnp.zeros((16,), jnp.float32)

    def zstep(i, _):
        zbuf[pl.ds(i * 16, 16)] = zero
        return 0
    lax.fori_loop(0, 40, zstep, 0)
    pltpu.sync_copy(zbuf, sh_deg.at[pl.ds(sub * 640, 640)])

    @pl.when((core == 0) & (sub == 0))
    def _():
        pltpu.sync_copy(zbuf.at[pl.ds(0, CTOT)], sh_cval)
        pltpu.sync_copy(zbuf.at[pl.ds(0, CTOT)], sh_cgid)
        pltpu.sync_copy(zbuf.at[pl.ds(0, CTOT)], sh_cnt)
        pltpu.sync_copy(zbuf.at[pl.ds(0, CTOT)], sh_tvals)
        pltpu.sync_copy(zbuf.at[pl.ds(0, CTOT)], sh_perm)

    plsc.subcore_barrier()

    pltpu.sync_copy(dst2_hbm.at[pl.ds(wid * RPW, RPW)], dst_t)
    pltpu.sync_copy(w2_hbm.at[pl.ds(wid * RPW, RPW)], w_t)

    def dstep(j, _):
        pltpu.sync_copy(w_t.at[j], sh_deg.at[dst_t.at[j]], add=True)
        return 0
    lax.fori_loop(0, RPW, dstep, 0)

    @pl.when(core == 0)
    def _():
        pltpu.sync_copy(sval_hbm, svalv)
        pltpu.sync_copy(slot_hbm, slotv)
        for g in range(8):
            onesb[pl.ds(g * 16, 16)] = one
        subf = jnp.broadcast_to(sub, (16,)).astype(jnp.float32)
        for r in range(5):
            basef = (subf * 5.0 + float(r)) * 128.0
            for g in range(8):
                gidb[r, pl.ds(g * 16, 16)] = (
                    basef + lane_f + float(g * 16))
        for r in range(5):
            rw = sub * 5 + r
            pltpu.sync_copy(svalv.at[rw], sh_cval.at[slotv.at[rw]], add=True)
            pltpu.sync_copy(gidb.at[r], sh_cgid.at[slotv.at[rw]], add=True)
            pltpu.sync_copy(onesb, sh_cnt.at[slotv.at[rw]], add=True)

    plsc.subcore_barrier()

    pltpu.sync_copy(sh_deg.at[pl.ds(sub * 640, 640)],
                    degp_out.at[core, pl.ds(sub * 640, 640)])

    @pl.when(core == 0)
    def _():
        pltpu.sync_copy(sh_cval, cval_t)
        pltpu.sync_copy(sh_cgid, cgid_t)
        pltpu.sync_copy(sh_cnt, cnt_t)
        bigneg = jnp.full((16,), BIG_NEG, jnp.float32)
        mycnt = cnt_t[pl.ds(sub * 16, 16)]
        myval = jnp.where(mycnt > 0.5, cval_t[pl.ds(sub * 16, 16)], bigneg)
        mygid = cgid_t[pl.ds(sub * 16, 16)]
        vbuf[...] = myval
        gbuf[...] = mygid
        myval = vbuf[...]
        mygid = gbuf[...]

        rfin = zero
        for half in range(2):
            accs = [zero] * 8
            kls = [jnp.broadcast_to(myval[half * 8 + l], (16,))
                   for l in range(8)]
            gls = [jnp.broadcast_to(mygid[half * 8 + l], (16,))
                   for l in range(8)]

            def scan(v2, accs):
                sv0 = cval_t[pl.ds(v2 * 16, 16)]
                cn = cnt_t[pl.ds(v2 * 16, 16)]
                sv = jnp.where(cn > 0.5, sv0, bigneg)
                gv = cgid_t[pl.ds(v2 * 16, 16)]
                out = []
                for l in range(8):
                    gt = jnp.where(sv > kls[l], one, zero)
                    eq = jnp.where(sv == kls[l], one, zero)
                    lt = jnp.where(gv < gls[l], one, zero)
                    out.append(accs[l] + gt + eq * lt)
                return out
            accs = lax.fori_loop(0, CCAP // 16, scan, accs)
            for l in range(8):
                tot = jnp.zeros((16,), jnp.float32)
                for q in range(16):
                    tot = tot + jnp.broadcast_to(accs[l][q], (16,))
                sel = jnp.where(lane == (half * 8 + l), one, zero)
                rfin = rfin + tot * sel

        tbuf[...] = rfin
        rfin = tbuf[...]
        dbuf[...] = jnp.full((16,), float(K), jnp.float32) + lane_f + (
            jnp.broadcast_to(sub, (16,)).astype(jnp.float32) * 16.0)
        dump = dbuf[...]
        rcl = jnp.where(rfin < float(K), rfin, dump)
        ridx[...] = rcl.astype(jnp.int32)
        pltpu.sync_copy(vbuf, sh_tvals.at[ridx], add=True)
        pltpu.sync_copy(gbuf, sh_perm.at[ridx], add=True)

        plsc.subcore_barrier()

        @pl.when(sub < 8)
        def _():
            pltpu.sync_copy(sh_perm.at[pl.ds(0, K)], permf)
            for g in range(8):
                permi[pl.ds(g * 16, 16)] = (
                    permf[pl.ds(g * 16, 16)].astype(jnp.int32))
            pltpu.async_copy(x_hbm.at[permi.at[pl.ds(sub * 16, 16)]],
                             xrows, gsem).wait()
            pltpu.sync_copy(xrows, xsel_out.at[pl.ds(sub * 16, 16)])

        @pl.when(sub == 15)
        def _():
            pltpu.sync_copy(sh_tvals.at[pl.ds(0, K)], tvals_out)


_SC_SELECT_OUT = [
    jax.ShapeDtypeStruct((K, D), jnp.float32),
    jax.ShapeDtypeStruct((K,), jnp.float32),
    jax.ShapeDtypeStruct((2, NPAD), jnp.float32),
]
_SC_SELECT_SCRATCH = [
    pltpu.VMEM((RPW, 128), jnp.int32),
    pltpu.VMEM((RPW, 128), jnp.float32),
    pltpu.VMEM((640,), jnp.float32),
    pltpu.VMEM((NB, 128), jnp.float32),
    pltpu.VMEM((NB, 128), jnp.int32),
    pltpu.VMEM((5, 128), jnp.float32),
    pltpu.VMEM((128,), jnp.float32),
    pltpu.VMEM((CTOT,), jnp.float32),
    pltpu.VMEM((CTOT,), jnp.float32),
    pltpu.VMEM((CTOT,), jnp.float32),
    pltpu.VMEM((16,), jnp.int32),
    pltpu.VMEM((16,), jnp.float32),
    pltpu.VMEM((16,), jnp.float32),
    pltpu.VMEM((16,), jnp.float32),
    pltpu.VMEM((16,), jnp.float32),
    pltpu.VMEM((K,), jnp.float32),
    pltpu.VMEM((K,), jnp.int32),
    pltpu.VMEM((16, D), jnp.float32),
    pltpu.VMEM_SHARED((NPAD,), jnp.float32),
    pltpu.VMEM_SHARED((CTOT,), jnp.float32),
    pltpu.VMEM_SHARED((CTOT,), jnp.float32),
    pltpu.VMEM_SHARED((CTOT,), jnp.float32),
    pltpu.VMEM_SHARED((CTOT,), jnp.float32),
    pltpu.VMEM_SHARED((CTOT,), jnp.float32),
    pltpu.SemaphoreType.DMA,
]


def _evolve_body(xsel_ref, tv_ref, Wgcn_ref, Wih_ref, Whh_ref, bih_ref,
                 bhh_ref, x_ref, d0_ref, d1_ref, y_ref, dinv_ref):
    xs = xsel_ref[...]
    tv = tv_ref[...]
    xt = xs * jnp.tanh(tv)
    dn = (((1,), (1,)), ((), ()))
    bf = jnp.bfloat16
    gi = lax.dot_general(xt.astype(bf), Wih_ref[...].astype(bf), dn,
                         preferred_element_type=jnp.float32) + bih_ref[...]
    gh = lax.dot_general(Wgcn_ref[...].astype(bf), Whh_ref[...].astype(bf),
                         dn,
                         preferred_element_type=jnp.float32) + bhh_ref[...]
    r = jax.nn.sigmoid(gi[:, :D] + gh[:, :D])
    z = jax.nn.sigmoid(gi[:, D:2 * D] + gh[:, D:2 * D])
    n = jnp.tanh(gi[:, 2 * D:] + r * gh[:, 2 * D:])
    w_ev = (1.0 - z) * n + z * Wgcn_ref[...]
    xw = lax.dot_general(x_ref[...].astype(bf), w_ev.astype(bf),
                         (((1,), (0,)), ((), ())),
                         preferred_element_type=jnp.float32)
    deg = 1.0 + d0_ref[...] + d1_ref[...]
    dinv = lax.rsqrt(deg)
    y_ref[...] = xw * dinv
    dinv_ref[...] = dinv


def _evolve(xsel, tvals2, w_gcn, w_ih, w_hh, b_ih2, b_hh2, x2, d0, d1):
    return pl.pallas_call(
        _evolve_body,
        out_shape=[
            jax.ShapeDtypeStruct((NPAD, D), jnp.float32),
            jax.ShapeDtypeStruct((NPAD, 1), jnp.float32),
        ],
    )(xsel, tvals2, w_gcn, w_ih, w_hh, b_ih2, b_hh2, x2, d0, d1)


def _scatter_body(y_hbm, src2_hbm, dst2_hbm, w2_hbm, acc_out,
                  src_t, dst_t, w_t, rows, zrows, sh_acc, gsem):
    core = lax.axis_index("c")
    sub = lax.axis_index("s")
    wid = sub * 2 + core

    zero = jnp.zeros((16,), jnp.float32)
    for i in range(8):
        for u in range(8):
            zrows[i, pl.ds(u * 16, 16)] = zero
    for q in range(79):
        pltpu.sync_copy(zrows, sh_acc.at[pl.ds(sub * NRW + q * 8, 8)])
    plsc.subcore_barrier()

    def estep(j, _):
        pltpu.async_copy(y_hbm.at[src_t.at[j]], rows, gsem).wait()
        for g in range(8):
            wv = w_t[j, pl.ds(g * 16, 16)]
            for l in range(16):
                rr = g * 16 + l
                wsp = jnp.broadcast_to(wv[l], (16,))
                for u in range(8):
                    rows[rr, pl.ds(u * 16, 16)] = (
                        rows[rr, pl.ds(u * 16, 16)] * wsp)
        pltpu.sync_copy(rows, sh_acc.at[dst_t.at[j]], add=True)
        return 0

    for c in range(5):
        pltpu.sync_copy(src2_hbm.at[pl.ds(wid * RPW + c * 16, 16)], src_t)
        pltpu.sync_copy(dst2_hbm.at[pl.ds(wid * RPW + c * 16, 16)], dst_t)
        pltpu.sync_copy(w2_hbm.at[pl.ds(wid * RPW + c * 16, 16)], w_t)
        lax.fori_loop(0, 16, estep, 0)

    plsc.subcore_barrier()
    pltpu.sync_copy(sh_acc.at[pl.ds(sub * NRW, NRW)],
                    acc_out.at[core, pl.ds(sub * NRW, NRW)])


_SC_SCATTER_OUT = jax.ShapeDtypeStruct((2, NACC, D), jnp.float32)
_SC_SCATTER_SCRATCH = [
    pltpu.VMEM((16, 128), jnp.int32),
    pltpu.VMEM((16, 128), jnp.int32),
    pltpu.VMEM((16, 128), jnp.float32),
    pltpu.VMEM((128, 128), jnp.float32),
    pltpu.VMEM((8, 128), jnp.float32),
    pltpu.VMEM_SHARED((NACC, D), jnp.float32),
    pltpu.SemaphoreType.DMA,
]


@functools.cache
def _sc_kernels():
    mesh = plsc.VectorSubcoreMesh(core_axis_name="c", subcore_axis_name="s")
    sel = pl.kernel(_select_body, out_type=_SC_SELECT_OUT, mesh=mesh,
                    scratch_types=_SC_SELECT_SCRATCH)
    sca = pl.kernel(_scatter_body, out_type=_SC_SCATTER_OUT, mesh=mesh,
                    scratch_types=_SC_SCATTER_SCRATCH)
    return sel, sca


def _head_body(a_ref, y_ref, dinv_ref, wlin_ref, blin_ref, out_ref):
    s = a_ref[0, :N] + a_ref[1, :N]
    h = jnp.maximum(dinv_ref[...] * (s + y_ref[...]), 0.0)
    bf = jnp.bfloat16
    out_ref[...] = lax.dot_general(
        h.astype(bf), wlin_ref[...].astype(bf), (((1,), (0,)), ((), ())),
        preferred_element_type=jnp.float32) + blin_ref[...]


def _head(acc, y10k, dinv10k, w_lin, b_lin2):
    return pl.pallas_call(
        _head_body,
        out_shape=jax.ShapeDtypeStruct((N, 1), jnp.float32),
    )(acc, y10k, dinv10k, w_lin, b_lin2)


def kernel(x, edge_index, edge_weight, p, W_ih, W_hh, b_ih, b_hh, W_gcn,
           W_lin, b_lin):
    f32 = jnp.float32
    x_pad = jnp.concatenate([x, jnp.zeros((NPAD - N, D), f32)], axis=0)
    x3 = x_pad.reshape(NB, D, D)
    p2 = p.reshape(1, D)

    src = edge_index[0].astype(jnp.int32)
    dst = edge_index[1].astype(jnp.int32)
    e = src.shape[0]
    padn = EPAD - e
    pidx = (jnp.arange(padn, dtype=jnp.int32)) % N
    src2 = jnp.concatenate([src, pidx]).reshape(EROWS, 128)
    dst2 = jnp.concatenate([dst, pidx]).reshape(EROWS, 128)
    w2 = jnp.concatenate([edge_weight.astype(f32),
                          jnp.zeros((padn,), f32)]).reshape(EROWS, 128)

    nrm_b = jnp.broadcast_to(jnp.linalg.norm(p), (NB, D))
    sval, slot = _scores_and_slots(x3, p2, nrm_b)

    sc_select, sc_scatter = _sc_kernels()
    xsel, tvals, degp = sc_select(sval, slot, x_pad, dst2, w2)

    d0 = degp[0].reshape(NPAD, 1)
    d1 = degp[1].reshape(NPAD, 1)
    y, dinv = _evolve(xsel, tvals.reshape(K, 1), W_gcn, W_ih, W_hh,
                      b_ih.reshape(1, 3 * D), b_hh.reshape(1, 3 * D),
                      x_pad, d0, d1)

    acc = sc_scatter(y, src2, dst2, w2)

    b_col = jnp.broadcast_to(b_lin.reshape(1, 1), (N, 1))
    out = _head(acc, y[:N], dinv[:N], W_lin.reshape(D, 1), b_col)
    return out


# speedup vs baseline: 29.9470x; 1.3588x over previous
import functools

import jax
import jax.numpy as jnp
from jax import lax
from jax.experimental import pallas as pl
from jax.experimental.pallas import tpu as pltpu
from jax.experimental.pallas import tpu_sc as plsc

N = 10000
D = 128
K = 128
NB = 80
NPAD = NB * 128
EROWS = 2560
EPAD = EROWS * 128
RPW = EROWS // 32
NACC = 10112
NRW = NACC // 16
CCAP = 256
CTOT = 384
BIG_NEG = -3.0e38


HI = lax.Precision.HIGHEST


def _score_body(x3_ref, p_ref, nrm_ref, sval_ref, slot_ref):
    x3 = x3_ref[...].astype(jnp.bfloat16)
    pp = p_ref[...].astype(jnp.bfloat16)
    s3 = lax.dot_general(x3, pp, (((2,), (1,)), ((), ())),
                         preferred_element_type=jnp.float32)
    s = s3[:, :, 0] / nrm_ref[...]
    ii = lax.broadcasted_iota(jnp.int32, (NB, D), 0)
    jj = lax.broadcasted_iota(jnp.int32, (NB, D), 1)
    valid = ii * D + jj < N
    s = jnp.where(valid, s, -jnp.inf)
    imin = jnp.int32(-(2 ** 31))
    b = lax.bitcast_convert_type(s, jnp.int32)
    skey = b ^ (lax.shift_right_arithmetic(b, 31) & jnp.int32(0x7FFFFFFF))
    skey = jnp.where(valid, skey, imin)

    def step(i, pref):
        cand = pref | lax.shift_left(jnp.int32(1), 31 - i)
        scand = cand ^ imin
        cnt = jnp.sum((skey >= scand).astype(jnp.int32))
        return jnp.where(cnt >= K, cand, pref)

    pref = lax.fori_loop(0, 32, step, jnp.int32(0))
    thr_f = lax.bitcast_convert_type(pref ^ imin, jnp.float32)
    flag = s >= thr_f
    flag_f = jnp.where(flag, 1.0, 0.0).astype(jnp.float32)
    u0 = lax.broadcasted_iota(jnp.int32, (D, D), 0)
    u1 = lax.broadcasted_iota(jnp.int32, (D, D), 1)
    tri_u = jnp.where(u0 < u1, 1.0, 0.0).astype(jnp.float32)
    v0 = lax.broadcasted_iota(jnp.int32, (NB, NB), 0)
    v1 = lax.broadcasted_iota(jnp.int32, (NB, NB), 1)
    tri_l = jnp.where(v1 < v0, 1.0, 0.0).astype(jnp.float32)
    dnum = (((1,), (0,)), ((), ()))
    ex_lane = lax.dot_general(flag_f, tri_u, dnum, precision=HI,
                              preferred_element_type=jnp.float32)
    ones_col = jnp.full((D, 1), 1.0, jnp.float32)
    row_tot = lax.dot_general(flag_f, ones_col, dnum, precision=HI,
                              preferred_element_type=jnp.float32)
    row_pre = lax.dot_general(tri_l, row_tot, dnum, precision=HI,
                              preferred_element_type=jnp.float32)
    pos = ex_lane + row_pre
    slot = jnp.where(flag, jnp.minimum(pos, float(CCAP - 1)).astype(jnp.int32),
                     CCAP + jj)
    sval_ref[...] = s
    slot_ref[...] = slot


def _scores_and_slots(x3, p2, nrm_b):
    return pl.pallas_call(
        _score_body,
        out_shape=[
            jax.ShapeDtypeStruct((NB, D), jnp.float32),
            jax.ShapeDtypeStruct((NB, D), jnp.int32),
        ],
    )(x3, p2, nrm_b)


def _select_body(sval_hbm, slot_hbm, x_hbm, dst2_hbm, w2_hbm,
                 xsel_out, tvals_out, degp_out,
                 dst_t, w_t, zbuf, svalv, slotv, gidb, onesb,
                 cval_t, cgid_t, cnt_t, ridx, tbuf, vbuf, gbuf, dbuf,
                 permf, permi, xrows, sh_deg, sh_cval, sh_cgid, sh_cnt,
                 sh_tvals, sh_perm, gsem):
    core = lax.axis_index("c")
    sub = lax.axis_index("s")
    wid = sub * 2 + core
    lane = lax.iota(jnp.int32, 16)
    lane_f = lane.astype(jnp.float32)
    one = jnp.full((16,), 1.0, jnp.float32)
    zero = jnp.zeros((16,), jnp.float32)

    def zstep(i, _):
        zbuf[pl.ds(i * 16, 16)] = zero
        return 0
    lax.fori_loop(0, 40, zstep, 0)
    pltpu.sync_copy(zbuf, sh_deg.at[pl.ds(sub * 640, 640)])

    @pl.when((core == 0) & (sub == 0))
    def _():
        pltpu.sync_copy(zbuf.at[pl.ds(0, CTOT)], sh_cval)
        pltpu.sync_copy(zbuf.at[pl.ds(0, CTOT)], sh_cgid)
        pltpu.sync_copy(zbuf.at[pl.ds(0, CTOT)], sh_cnt)
        pltpu.sync_copy(zbuf.at[pl.ds(0, CTOT)], sh_tvals)
        pltpu.sync_copy(zbuf.at[pl.ds(0, CTOT)], sh_perm)

    plsc.subcore_barrier()

    pltpu.sync_copy(dst2_hbm.at[pl.ds(wid * RPW, RPW)], dst_t)
    pltpu.sync_copy(w2_hbm.at[pl.ds(wid * RPW, RPW)], w_t)

    def dstep(j, _):
        pltpu.sync_copy(w_t.at[j], sh_deg.at[dst_t.at[j]], add=True)
        return 0
    lax.fori_loop(0, RPW, dstep, 0)

    @pl.when(core == 0)
    def _():
        pltpu.sync_copy(sval_hbm, svalv)
        pltpu.sync_copy(slot_hbm, slotv)
        for g in range(8):
            onesb[pl.ds(g * 16, 16)] = one
        subf = jnp.broadcast_to(sub, (16,)).astype(jnp.float32)
        for r in range(5):
            basef = (subf * 5.0 + float(r)) * 128.0
            for g in range(8):
                gidb[r, pl.ds(g * 16, 16)] = (
                    basef + lane_f + float(g * 16))
        for r in range(5):
            rw = sub * 5 + r
            pltpu.sync_copy(svalv.at[rw], sh_cval.at[slotv.at[rw]], add=True)
            pltpu.sync_copy(gidb.at[r], sh_cgid.at[slotv.at[rw]], add=True)
            pltpu.sync_copy(onesb, sh_cnt.at[slotv.at[rw]], add=True)

    plsc.subcore_barrier()

    pltpu.sync_copy(sh_deg.at[pl.ds(sub * 640, 640)],
                    degp_out.at[core, pl.ds(sub * 640, 640)])

    @pl.when(core == 0)
    def _():
        pltpu.sync_copy(sh_cval, cval_t)
        pltpu.sync_copy(sh_cgid, cgid_t)
        pltpu.sync_copy(sh_cnt, cnt_t)
        bigneg = jnp.full((16,), BIG_NEG, jnp.float32)
        mycnt = cnt_t[pl.ds(sub * 16, 16)]
        myval = jnp.where(mycnt > 0.5, cval_t[pl.ds(sub * 16, 16)], bigneg)
        mygid = cgid_t[pl.ds(sub * 16, 16)]
        vbuf[...] = myval
        gbuf[...] = mygid
        myval = vbuf[...]
        mygid = gbuf[...]

        rfin = zero
        for half in range(2):
            accs = [zero] * 8
            kls = [jnp.broadcast_to(myval[half * 8 + l], (16,))
                   for l in range(8)]
            gls = [jnp.broadcast_to(mygid[half * 8 + l], (16,))
                   for l in range(8)]

            def scan(v2, accs):
                sv0 = cval_t[pl.ds(v2 * 16, 16)]
                cn = cnt_t[pl.ds(v2 * 16, 16)]
                sv = jnp.where(cn > 0.5, sv0, bigneg)
                gv = cgid_t[pl.ds(v2 * 16, 16)]
                out = []
                for l in range(8):
                    gt = jnp.where(sv > kls[l], one, zero)
                    eq = jnp.where(sv == kls[l], one, zero)
                    lt = jnp.where(gv < gls[l], one, zero)
                    out.append(accs[l] + gt + eq * lt)
                return out
            accs = lax.fori_loop(0, CCAP // 16, scan, accs)
            for l in range(8):
                tot = jnp.zeros((16,), jnp.float32)
                for q in range(16):
                    tot = tot + jnp.broadcast_to(accs[l][q], (16,))
                sel = jnp.where(lane == (half * 8 + l), one, zero)
                rfin = rfin + tot * sel

        tbuf[...] = rfin
        rfin = tbuf[...]
        dbuf[...] = jnp.full((16,), float(K), jnp.float32) + lane_f + (
            jnp.broadcast_to(sub, (16,)).astype(jnp.float32) * 16.0)
        dump = dbuf[...]
        rcl = jnp.where(rfin < float(K), rfin, dump)
        ridx[...] = rcl.astype(jnp.int32)
        pltpu.sync_copy(vbuf, sh_tvals.at[ridx], add=True)
        pltpu.sync_copy(gbuf, sh_perm.at[ridx], add=True)

        plsc.subcore_barrier()

        @pl.when(sub < 8)
        def _():
            pltpu.sync_copy(sh_perm.at[pl.ds(0, K)], permf)
            for g in range(8):
                permi[pl.ds(g * 16, 16)] = (
                    permf[pl.ds(g * 16, 16)].astype(jnp.int32))
            pltpu.async_copy(x_hbm.at[permi.at[pl.ds(sub * 16, 16)]],
                             xrows, gsem).wait()
            pltpu.sync_copy(xrows, xsel_out.at[pl.ds(sub * 16, 16)])

        @pl.when(sub == 15)
        def _():
            pltpu.sync_copy(sh_tvals.at[pl.ds(0, K)], tvals_out)


_SC_SELECT_OUT = [
    jax.ShapeDtypeStruct((K, D), jnp.float32),
    jax.ShapeDtypeStruct((K,), jnp.float32),
    jax.ShapeDtypeStruct((2, NPAD), jnp.float32),
]
_SC_SELECT_SCRATCH = [
    pltpu.VMEM((RPW, 128), jnp.int32),
    pltpu.VMEM((RPW, 128), jnp.float32),
    pltpu.VMEM((640,), jnp.float32),
    pltpu.VMEM((NB, 128), jnp.float32),
    pltpu.VMEM((NB, 128), jnp.int32),
    pltpu.VMEM((5, 128), jnp.float32),
    pltpu.VMEM((128,), jnp.float32),
    pltpu.VMEM((CTOT,), jnp.float32),
    pltpu.VMEM((CTOT,), jnp.float32),
    pltpu.VMEM((CTOT,), jnp.float32),
    pltpu.VMEM((16,), jnp.int32),
    pltpu.VMEM((16,), jnp.float32),
    pltpu.VMEM((16,), jnp.float32),
    pltpu.VMEM((16,), jnp.float32),
    pltpu.VMEM((16,), jnp.float32),
    pltpu.VMEM((K,), jnp.float32),
    pltpu.VMEM((K,), jnp.int32),
    pltpu.VMEM((16, D), jnp.float32),
    pltpu.VMEM_SHARED((NPAD,), jnp.float32),
    pltpu.VMEM_SHARED((CTOT,), jnp.float32),
    pltpu.VMEM_SHARED((CTOT,), jnp.float32),
    pltpu.VMEM_SHARED((CTOT,), jnp.float32),
    pltpu.VMEM_SHARED((CTOT,), jnp.float32),
    pltpu.VMEM_SHARED((CTOT,), jnp.float32),
    pltpu.SemaphoreType.DMA,
]


def _evolve_body(xsel_ref, tv_ref, Wgcn_ref, Wih_ref, Whh_ref, bih_ref,
                 bhh_ref, x_ref, d0_ref, d1_ref, y_ref, dinv_ref):
    xs = xsel_ref[...]
    tv = tv_ref[...]
    xt = xs * jnp.tanh(tv)
    dn = (((1,), (1,)), ((), ()))
    bf = jnp.bfloat16
    gi = lax.dot_general(xt.astype(bf), Wih_ref[...].astype(bf), dn,
                         preferred_element_type=jnp.float32) + bih_ref[...]
    gh = lax.dot_general(Wgcn_ref[...].astype(bf), Whh_ref[...].astype(bf),
                         dn,
                         preferred_element_type=jnp.float32) + bhh_ref[...]
    r = jax.nn.sigmoid(gi[:, :D] + gh[:, :D])
    z = jax.nn.sigmoid(gi[:, D:2 * D] + gh[:, D:2 * D])
    n = jnp.tanh(gi[:, 2 * D:] + r * gh[:, 2 * D:])
    w_ev = (1.0 - z) * n + z * Wgcn_ref[...]
    xw = lax.dot_general(x_ref[...].astype(bf), w_ev.astype(bf),
                         (((1,), (0,)), ((), ())),
                         preferred_element_type=jnp.float32)
    deg = 1.0 + d0_ref[...] + d1_ref[...]
    dinv = lax.rsqrt(deg)
    y_ref[...] = xw * dinv
    dinv_ref[...] = dinv


def _evolve(xsel, tvals2, w_gcn, w_ih, w_hh, b_ih2, b_hh2, x2, d0, d1):
    return pl.pallas_call(
        _evolve_body,
        out_shape=[
            jax.ShapeDtypeStruct((NPAD, D), jnp.float32),
            jax.ShapeDtypeStruct((NPAD, 1), jnp.float32),
        ],
    )(xsel, tvals2, w_gcn, w_ih, w_hh, b_ih2, b_hh2, x2, d0, d1)


def _scatter_body(y_hbm, src2_hbm, dst2_hbm, w2_hbm, acc_out,
                  src_t, dst_t, w_t, rows, rows2, zrows, sh_acc, gsem,
                  gsem2):
    core = lax.axis_index("c")
    sub = lax.axis_index("s")
    wid = sub * 2 + core

    zero = jnp.zeros((16,), jnp.float32)
    for i in range(8):
        for u in range(8):
            zrows[i, pl.ds(u * 16, 16)] = zero
    for q in range(79):
        pltpu.sync_copy(zrows, sh_acc.at[pl.ds(sub * NRW + q * 8, 8)])
    plsc.subcore_barrier()

    def scale_scatter(buf, j):
        def sgroup(g, _):
            wv = w_t[j, pl.ds(g * 16, 16)]
            for l in range(16):
                wsp = jnp.broadcast_to(wv[l], (16,))
                for u in range(8):
                    buf[g * 16 + l, pl.ds(u * 16, 16)] = (
                        buf[g * 16 + l, pl.ds(u * 16, 16)] * wsp)
            return 0
        lax.fori_loop(0, 8, sgroup, 0)
        pltpu.sync_copy(buf, sh_acc.at[dst_t.at[j]], add=True)

    for c in range(5):
        pltpu.sync_copy(src2_hbm.at[pl.ds(wid * RPW + c * 16, 16)], src_t)
        pltpu.sync_copy(dst2_hbm.at[pl.ds(wid * RPW + c * 16, 16)], dst_t)
        pltpu.sync_copy(w2_hbm.at[pl.ds(wid * RPW + c * 16, 16)], w_t)
        pltpu.async_copy(y_hbm.at[src_t.at[0]], rows, gsem)

        def pstep(i, _):
            j0 = 2 * i
            pltpu.make_async_copy(y_hbm.at[src_t.at[j0]], rows, gsem).wait()
            pltpu.async_copy(y_hbm.at[src_t.at[j0 + 1]], rows2, gsem2)
            scale_scatter(rows, j0)
            pltpu.make_async_copy(y_hbm.at[src_t.at[j0 + 1]], rows2,
                                  gsem2).wait()

            @pl.when(i < 7)
            def _():
                pltpu.async_copy(y_hbm.at[src_t.at[j0 + 2]], rows, gsem)
            scale_scatter(rows2, j0 + 1)
            return 0
        lax.fori_loop(0, 8, pstep, 0)

    plsc.subcore_barrier()
    pltpu.sync_copy(sh_acc.at[pl.ds(sub * NRW, NRW)],
                    acc_out.at[core, pl.ds(sub * NRW, NRW)])


_SC_SCATTER_OUT = jax.ShapeDtypeStruct((2, NACC, D), jnp.float32)
_SC_SCATTER_SCRATCH = [
    pltpu.VMEM((16, 128), jnp.int32),
    pltpu.VMEM((16, 128), jnp.int32),
    pltpu.VMEM((16, 128), jnp.float32),
    pltpu.VMEM((128, 128), jnp.float32),
    pltpu.VMEM((128, 128), jnp.float32),
    pltpu.VMEM((8, 128), jnp.float32),
    pltpu.VMEM_SHARED((NACC, D), jnp.float32),
    pltpu.SemaphoreType.DMA,
    pltpu.SemaphoreType.DMA,
]


@functools.cache
def _sc_kernels():
    mesh = plsc.VectorSubcoreMesh(core_axis_name="c", subcore_axis_name="s")
    sel = pl.kernel(_select_body, out_type=_SC_SELECT_OUT, mesh=mesh,
                    scratch_types=_SC_SELECT_SCRATCH)
    sca = pl.kernel(_scatter_body, out_type=_SC_SCATTER_OUT, mesh=mesh,
                    scratch_types=_SC_SCATTER_SCRATCH)
    return sel, sca


def _head_body(a_ref, y_ref, dinv_ref, wlin_ref, blin_ref, out_ref):
    s = a_ref[0, :N] + a_ref[1, :N]
    h = jnp.maximum(dinv_ref[...] * (s + y_ref[...]), 0.0)
    bf = jnp.bfloat16
    out_ref[...] = lax.dot_general(
        h.astype(bf), wlin_ref[...].astype(bf), (((1,), (0,)), ((), ())),
        preferred_element_type=jnp.float32) + blin_ref[...]


def _head(acc, y10k, dinv10k, w_lin, b_lin2):
    return pl.pallas_call(
        _head_body,
        out_shape=jax.ShapeDtypeStruct((N, 1), jnp.float32),
    )(acc, y10k, dinv10k, w_lin, b_lin2)


def kernel(x, edge_index, edge_weight, p, W_ih, W_hh, b_ih, b_hh, W_gcn,
           W_lin, b_lin):
    f32 = jnp.float32
    x_pad = jnp.concatenate([x, jnp.zeros((NPAD - N, D), f32)], axis=0)
    x3 = x_pad.reshape(NB, D, D)
    p2 = p.reshape(1, D)

    src = edge_index[0].astype(jnp.int32)
    dst = edge_index[1].astype(jnp.int32)
    e = src.shape[0]
    padn = EPAD - e
    pidx = (jnp.arange(padn, dtype=jnp.int32)) % N
    src2 = jnp.concatenate([src, pidx]).reshape(EROWS, 128)
    dst2 = jnp.concatenate([dst, pidx]).reshape(EROWS, 128)
    w2 = jnp.concatenate([edge_weight.astype(f32),
                          jnp.zeros((padn,), f32)]).reshape(EROWS, 128)

    nrm_b = jnp.broadcast_to(jnp.linalg.norm(p), (NB, D))
    sval, slot = _scores_and_slots(x3, p2, nrm_b)

    sc_select, sc_scatter = _sc_kernels()
    xsel, tvals, degp = sc_select(sval, slot, x_pad, dst2, w2)

    d0 = degp[0].reshape(NPAD, 1)
    d1 = degp[1].reshape(NPAD, 1)
    y, dinv = _evolve(xsel, tvals.reshape(K, 1), W_gcn, W_ih, W_hh,
                      b_ih.reshape(1, 3 * D), b_hh.reshape(1, 3 * D),
                      x_pad, d0, d1)

    acc = sc_scatter(y, src2, dst2, w2)

    b_col = jnp.broadcast_to(b_lin.reshape(1, 1), (N, 1))
    out = _head(acc, y[:N], dinv[:N], W_lin.reshape(D, 1), b_col)
    return out
